# baseline (speedup 1.0000x reference)
import functools

import jax
import jax.numpy as jnp
from jax import lax
from jax.experimental import pallas as pl
from jax.experimental.pallas import tpu as pltpu
from jax.experimental.pallas import tpu_sc as plsc

_N = 100000
_H = 64
_TXN_IN = 256

_CHUNK = 1024
_SUB = 128
_NPAD = 100352
_NCHUNKS = _NPAD // _CHUNK
_NW = 32

_sc_mesh = plsc.VectorSubcoreMesh(core_axis_name="c", subcore_axis_name="s")


@functools.partial(
    pl.kernel,
    mesh=_sc_mesh,
    out_type=[jax.ShapeDtypeStruct((_NPAD, _H), jnp.float32)] * 3,
    scratch_types=[
        pltpu.VMEM((_CHUNK // _SUB, _SUB), jnp.int32),
        pltpu.VMEM((_CHUNK, _H), jnp.float32),
        pltpu.SemaphoreType.DMA,
    ],
    compiler_params=pltpu.CompilerParams(use_tc_tiling_on_sc=False),
)
def _gather3(idx_c, idx_a, idx_e, mem_c, mem_a, mem_e,
             out_c, out_a, out_e, idx_v, rows_v, sem):
    wid = lax.axis_index("s") * 2 + lax.axis_index("c")
    for idx_hbm, mem_hbm, out_hbm in ((idx_c, mem_c, out_c),
                                      (idx_a, mem_a, out_a),
                                      (idx_e, mem_e, out_e)):
        for j in range((_NCHUNKS + _NW - 1) // _NW):
            c = wid + _NW * j

            @pl.when(c < _NCHUNKS)
            def _():
                row0 = c * (_CHUNK // _SUB)
                pltpu.sync_copy(idx_hbm.at[pl.ds(row0, _CHUNK // _SUB)], idx_v)
                cps = [
                    pltpu.async_copy(
                        mem_hbm.at[idx_v.at[k]],
                        rows_v.at[pl.ds(k * _SUB, _SUB)],
                        sem,
                    )
                    for k in range(_CHUNK // _SUB)
                ]
                for cp in cps:
                    cp.wait()
                pltpu.sync_copy(rows_v, out_hbm.at[pl.ds(c * _CHUNK, _CHUNK)])


_BR = 1000


def _mlp_body(x_ref, gc_ref, ga_ref, ge_ref, w1_ref, b1_ref,
              wv1_ref, bv1_ref, wv2_ref, bv2_ref, out_ref):
    x = x_ref[...]
    h = jnp.maximum(
        jnp.dot(x, w1_ref[...], preferred_element_type=jnp.float32) + b1_ref[...],
        0.0)
    acc = jnp.dot(h, wv1_ref[0:_H, :], preferred_element_type=jnp.float32)
    acc += jnp.dot(gc_ref[...], wv1_ref[_H:2 * _H, :],
                   preferred_element_type=jnp.float32)
    acc += jnp.dot(ga_ref[...], wv1_ref[2 * _H:3 * _H, :],
                   preferred_element_type=jnp.float32)
    acc += jnp.dot(ge_ref[...], wv1_ref[3 * _H:4 * _H, :],
                   preferred_element_type=jnp.float32)
    z = jnp.maximum(acc + bv1_ref[...], 0.0)
    out_ref[...] = (
        jnp.dot(z, wv2_ref[...], preferred_element_type=jnp.float32)
        + bv2_ref[...])


def _mlp(txn_x, gc, ga, ge, W1, b1, Wv1, bv1, Wv2, bv2):
    grid = _N // _BR
    return pl.pallas_call(
        _mlp_body,
        grid=(grid,),
        in_specs=[
            pl.BlockSpec((_BR, _TXN_IN), lambda i: (i, 0)),
            pl.BlockSpec((_BR, _H), lambda i: (i, 0)),
            pl.BlockSpec((_BR, _H), lambda i: (i, 0)),
            pl.BlockSpec((_BR, _H), lambda i: (i, 0)),
            pl.BlockSpec((_TXN_IN, _H), lambda i: (0, 0)),
            pl.BlockSpec((1, _H), lambda i: (0, 0)),
            pl.BlockSpec((4 * _H, _H), lambda i: (0, 0)),
            pl.BlockSpec((1, _H), lambda i: (0, 0)),
            pl.BlockSpec((_H, 1), lambda i: (0, 0)),
            pl.BlockSpec((1, 1), lambda i: (0, 0)),
        ],
        out_specs=pl.BlockSpec((_BR, 1), lambda i: (i, 0)),
        out_shape=jax.ShapeDtypeStruct((_N, 1), jnp.float32),
        compiler_params=pltpu.CompilerParams(
            dimension_semantics=("arbitrary",),
        ),
    )(txn_x, gc, ga, ge, W1, b1, Wv1, bv1, Wv2, bv2)


def kernel(txn_x, idx_card, idx_addr, idx_email, mem_card, mem_addr, mem_email,
           W1, b1, unk_card, unk_addr, unk_email, Wv1, bv1, Wv2, bv2):
    pad = _NPAD - _N
    idx2d = [
        jnp.pad(i.astype(jnp.int32), (0, pad)).reshape(_NPAD // _SUB, _SUB)
        for i in (idx_card, idx_addr, idx_email)
    ]
    gc, ga, ge = _gather3(idx2d[0], idx2d[1], idx2d[2],
                          mem_card, mem_addr, mem_email)
    out = _mlp(txn_x, gc, ga, ge,
               W1, b1.reshape(1, _H), Wv1, bv1.reshape(1, _H),
               Wv2, bv2.reshape(1, 1))
    return out[:, 0]
